# Initial kernel scaffold; baseline (speedup 1.0000x reference)
import jax
import jax.numpy as jnp
from jax.experimental import pallas as pl


def kernel(indexes, table, W):
    raise NotImplementedError("write your pallas kernel here")


# speedup vs baseline: 17.3403x; 17.3403x over previous
import functools

import jax
import jax.numpy as jnp
from jax import lax
from jax.experimental import pallas as pl
from jax.experimental.pallas import tpu as pltpu
from jax.experimental.pallas import tpu_sc as plsc

DIM = 32
NC = 2
NS = 16
NW = NC * NS

CHUNK = 128
K = 8


def _sc_gather(table, idx2d, total):
    n_per_w = total // NW
    rows_per_step = K * CHUNK
    steps = n_per_w // rows_per_step

    def body(table_hbm, idx_hbm, out_hbm, idx_v, rows_v, sem):
        wid = lax.axis_index("s") * NC + lax.axis_index("c")
        base = wid * n_per_w

        def step(c, _):
            off = pl.multiple_of(base + c * rows_per_step, rows_per_step)
            pltpu.sync_copy(idx_hbm.at[pl.ds(pl.multiple_of(off // CHUNK, K), K)], idx_v)
            cps = [
                pltpu.async_copy(
                    table_hbm.at[idx_v.at[j]],
                    rows_v.at[pl.ds(j * CHUNK, CHUNK)],
                    sem,
                )
                for j in range(K)
            ]
            for cp in cps:
                cp.wait()
            pltpu.sync_copy(rows_v, out_hbm.at[pl.ds(off, rows_per_step)])
            return ()

        lax.fori_loop(0, steps, step, ())

    kern = pl.kernel(
        body,
        out_type=jax.ShapeDtypeStruct((total, DIM), jnp.float32),
        mesh=plsc.VectorSubcoreMesh(core_axis_name="c", subcore_axis_name="s"),
        compiler_params=pltpu.CompilerParams(use_tc_tiling_on_sc=False),
        scratch_types=[
            pltpu.VMEM((K, CHUNK), jnp.int32),
            pltpu.VMEM((rows_per_step, DIM), jnp.float32),
            pltpu.SemaphoreType.DMA,
        ],
    )
    return kern(table, idx2d)


def _tc_project(x4, bd):
    n4 = x4.shape[0]
    blk = 1024

    def body(x_ref, w_ref, o_ref):
        o_ref[...] = jnp.dot(x_ref[...], w_ref[...],
                             preferred_element_type=jnp.float32)

    return pl.pallas_call(
        body,
        grid=(n4 // blk,),
        in_specs=[
            pl.BlockSpec((blk, 128), lambda i: (i, 0)),
            pl.BlockSpec((128, 128), lambda i: (0, 0)),
        ],
        out_specs=pl.BlockSpec((blk, 128), lambda i: (i, 0)),
        out_shape=jax.ShapeDtypeStruct((n4, 128), jnp.float32),
    )(x4, bd)


def kernel(indexes, table, W):
    B, L = indexes.shape
    total = B * L
    idx2d = indexes.reshape(-1).astype(jnp.int32).reshape(total // CHUNK, CHUNK)
    emb = _sc_gather(table, idx2d, total)
    bd = jnp.kron(jnp.eye(4, dtype=W.dtype), W.T)
    out4 = _tc_project(emb.reshape(total // 4, 128), bd)
    return out4.reshape(B, L, DIM)
